# baseline (speedup 1.0000x reference)
import functools

import jax
import jax.numpy as jnp
from jax import lax
from jax.experimental import pallas as pl
from jax.experimental.pallas import tpu as pltpu
from jax.experimental.pallas import tpu_sc as plsc

T = 4096
D_MODEL = 2048
HIDDEN = 1024
E = 16
TOPK = 2
SHARED_HIDDEN = 1024
CAPACITY = (T * TOPK // E) * 2
NROWS = E * CAPACITY
DUMP = NROWS
BUF_ROWS = NROWS + 8

TT = 512
BC = 128

NW = 32
TOK_PER_W = T // NW
DISP_CH = 32
COMB_CH = 8


def _route_body(x_ref, gw_ref, w0_ref, w1_ref, c0_ref, c1_ref, d0_ref, d1_ref,
                counts_ref):
    i = pl.program_id(0)

    @pl.when(i == 0)
    def _():
        counts_ref[...] = jnp.zeros_like(counts_ref)

    x = x_ref[...]
    gw = gw_ref[...]
    logits = lax.dot_general(x, gw, (((1,), (1,)), ((), ())),
                             preferred_element_type=jnp.float32,
                             precision=lax.Precision.DEFAULT)
    mx = jnp.max(logits, axis=1, keepdims=True)
    p = jnp.exp(logits - mx)
    s = p / jnp.sum(p, axis=1, keepdims=True)

    iota = lax.broadcasted_iota(jnp.int32, (TT, E), 1)
    m0 = jnp.max(s, axis=1, keepdims=True)
    a0 = jnp.min(jnp.where(s == m0, iota, E), axis=1)
    oh0 = iota == a0[:, None]
    s1 = jnp.where(oh0, -1.0, s)
    m1 = jnp.max(s1, axis=1, keepdims=True)
    a1 = jnp.min(jnp.where(s1 == m1, iota, E), axis=1)
    oh1 = iota == a1[:, None]

    oh0f = oh0.astype(jnp.float32)
    oh1f = oh1.astype(jnp.float32)
    oh = oh0f + oh1f
    r_i = lax.broadcasted_iota(jnp.int32, (TT, TT), 0)
    c_i = lax.broadcasted_iota(jnp.int32, (TT, TT), 1)
    tri = (r_i > c_i).astype(jnp.float32)
    S = lax.dot_general(tri, oh, (((1,), (0,)), ((), ())),
                        preferred_element_type=jnp.float32)
    base = counts_ref[0, :][None, :]
    tot = S + base
    pos0 = jnp.sum(tot * oh0f, axis=1).astype(jnp.int32)
    pos1 = jnp.sum(tot * oh1f, axis=1).astype(jnp.int32)
    counts_ref[0, :] = counts_ref[0, :] + jnp.sum(oh, axis=0)

    slot0 = jnp.minimum(pos0, CAPACITY - 1)
    slot1 = jnp.minimum(pos1, CAPACITY - 1)
    keep0 = pos0 < CAPACITY
    keep1 = pos1 < CAPACITY
    comb0 = a0 * CAPACITY + slot0
    comb1 = a1 * CAPACITY + slot1
    c0_ref[...] = comb0
    c1_ref[...] = comb1
    d0_ref[...] = jnp.where(keep0, comb0, DUMP)
    d1_ref[...] = jnp.where(keep1, comb1, DUMP)
    w0_ref[...] = jnp.broadcast_to(
        jnp.where(keep0, m0[:, 0], 0.0)[:, None], (TT, 16))
    w1_ref[...] = jnp.broadcast_to(
        jnp.where(keep1, m1[:, 0], 0.0)[:, None], (TT, 16))


def _route(x, gate_w):
    n = T // TT
    outs = (
        jax.ShapeDtypeStruct((T, 16), jnp.float32),
        jax.ShapeDtypeStruct((T, 16), jnp.float32),
        jax.ShapeDtypeStruct((T,), jnp.int32),
        jax.ShapeDtypeStruct((T,), jnp.int32),
        jax.ShapeDtypeStruct((T,), jnp.int32),
        jax.ShapeDtypeStruct((T,), jnp.int32),
    )
    vec_spec = pl.BlockSpec((TT,), lambda i: (i,))
    w_spec = pl.BlockSpec((TT, 16), lambda i: (i, 0))
    return pl.pallas_call(
        _route_body,
        grid=(n,),
        in_specs=[
            pl.BlockSpec((TT, D_MODEL), lambda i: (i, 0)),
            pl.BlockSpec((E, D_MODEL), lambda i: (0, 0)),
        ],
        out_specs=(w_spec, w_spec) + (vec_spec,) * 4,
        out_shape=outs,
        scratch_shapes=[pltpu.VMEM((1, E), jnp.float32)],
    )(x, gate_w)


DP_CH = 16
DP_NCH = TOK_PER_W // DP_CH


def _dispatch_body(x_hbm, d0_hbm, d1_hbm, buf_hbm,
                   xva, xvb, d0all, d1all, sxa, sxb, s0a, s0b, s1a, s1b):
    wid = lax.axis_index("s") * 2 + lax.axis_index("c")
    xv = [xva, xvb]
    sx = [sxa, sxb]
    ss0 = [s0a, s0b]
    ss1 = [s1a, s1b]
    pltpu.sync_copy(d0_hbm.at[wid], d0all)
    pltpu.sync_copy(d1_hbm.at[wid], d1all)

    def cp_in(cc, b):
        base = wid * TOK_PER_W + cc * DP_CH
        return pltpu.make_async_copy(x_hbm.at[pl.ds(base, DP_CH)], xv[b], sx[b])

    def sc0(cc, b):
        return pltpu.make_async_copy(xv[b], buf_hbm.at[d0all.at[cc]], ss0[b])

    def sc1(cc, b):
        return pltpu.make_async_copy(xv[b], buf_hbm.at[d1all.at[cc]], ss1[b])

    cp_in(0, 0).start()

    def outer(g, _):
        for b in range(2):
            cc = 2 * g + b
            b1 = 1 - b

            @pl.when(cc + 1 < DP_NCH)
            def _():
                @pl.when(cc >= 1)
                def _():
                    sc0(cc - 1, b1).wait()
                    sc1(cc - 1, b1).wait()

                cp_in(cc + 1, b1).start()

            cp_in(cc, b).wait()
            sc0(cc, b).start()
            sc1(cc, b).start()
        return ()

    lax.fori_loop(0, DP_NCH // 2, outer, ())
    sc0(DP_NCH - 2, 0).wait()
    sc1(DP_NCH - 2, 0).wait()
    sc0(DP_NCH - 1, 1).wait()
    sc1(DP_NCH - 1, 1).wait()


def _dispatch(x, d0, d1):
    mesh = plsc.VectorSubcoreMesh(core_axis_name="c", subcore_axis_name="s")
    f = pl.kernel(
        _dispatch_body,
        out_type=jax.ShapeDtypeStruct((BUF_ROWS, D_MODEL), jnp.float32),
        mesh=mesh,
        scratch_types=[
            pltpu.VMEM((DP_CH, D_MODEL), jnp.float32),
            pltpu.VMEM((DP_CH, D_MODEL), jnp.float32),
            pltpu.VMEM((DP_NCH, DP_CH), jnp.int32),
            pltpu.VMEM((DP_NCH, DP_CH), jnp.int32),
        ] + [pltpu.SemaphoreType.DMA] * 6,
    )
    return f(x, d0.reshape(NW, DP_NCH, DP_CH), d1.reshape(NW, DP_NCH, DP_CH))


def _experts_up_body(buf_ref, w1_ref, w3_ref, act_ref, w1s, w3s):
    @pl.when(pl.program_id(1) == 0)
    def _():
        w1s[...] = w1_ref[0].astype(jnp.bfloat16)
        w3s[...] = w3_ref[0].astype(jnp.bfloat16)

    xb = buf_ref[...].astype(jnp.bfloat16)
    h = lax.dot_general(xb, w1s[...], (((1,), (1,)), ((), ())),
                        preferred_element_type=jnp.float32)
    u = lax.dot_general(xb, w3s[...], (((1,), (1,)), ((), ())),
                        preferred_element_type=jnp.float32)
    act_ref[...] = (h * jax.nn.sigmoid(h) * u).astype(jnp.bfloat16)


def _experts_down_body(act_ref, w2_ref, ob_ref, w2s):
    @pl.when(pl.program_id(1) == 0)
    def _():
        w2s[...] = w2_ref[0].astype(jnp.bfloat16)

    ob_ref[...] = lax.dot_general(act_ref[...], w2s[...], (((1,), (1,)), ((), ())),
                                  preferred_element_type=jnp.float32)


BCA = 256
BCB = 512


def _experts(buf, w1, w3, w2):
    nca = CAPACITY // BCA
    act = pl.pallas_call(
        _experts_up_body,
        grid=(E, nca),
        in_specs=[
            pl.BlockSpec((BCA, D_MODEL), lambda e, c: (e * nca + c, 0)),
            pl.BlockSpec((1, HIDDEN, D_MODEL), lambda e, c: (e, 0, 0)),
            pl.BlockSpec((1, HIDDEN, D_MODEL), lambda e, c: (e, 0, 0)),
        ],
        out_specs=pl.BlockSpec((BCA, HIDDEN), lambda e, c: (e * nca + c, 0)),
        out_shape=jax.ShapeDtypeStruct((NROWS, HIDDEN), jnp.bfloat16),
        scratch_shapes=[pltpu.VMEM((HIDDEN, D_MODEL), jnp.bfloat16),
                        pltpu.VMEM((HIDDEN, D_MODEL), jnp.bfloat16)],
    )(buf, w1, w3)
    ncb = CAPACITY // BCB
    return pl.pallas_call(
        _experts_down_body,
        grid=(E, ncb),
        in_specs=[
            pl.BlockSpec((BCB, HIDDEN), lambda e, c: (e * ncb + c, 0)),
            pl.BlockSpec((1, D_MODEL, HIDDEN), lambda e, c: (e, 0, 0)),
        ],
        out_specs=pl.BlockSpec((BCB, D_MODEL), lambda e, c: (e * ncb + c, 0)),
        out_shape=jax.ShapeDtypeStruct((NROWS, D_MODEL), jnp.float32),
        scratch_shapes=[pltpu.VMEM((D_MODEL, HIDDEN), jnp.bfloat16)],
    )(act, w2)


def _shared_up_body(x_ref, sw1_ref, sw3_ref, act_ref, w1s, w3s):
    @pl.when(pl.program_id(0) == 0)
    def _():
        w1s[...] = sw1_ref[...].astype(jnp.bfloat16)
        w3s[...] = sw3_ref[...].astype(jnp.bfloat16)

    x = x_ref[...].astype(jnp.bfloat16)
    h = lax.dot_general(x, w1s[...], (((1,), (1,)), ((), ())),
                        preferred_element_type=jnp.float32)
    u = lax.dot_general(x, w3s[...], (((1,), (1,)), ((), ())),
                        preferred_element_type=jnp.float32)
    act_ref[...] = (h * jax.nn.sigmoid(h) * u).astype(jnp.bfloat16)


def _shared_down_body(act_ref, sw2_ref, y_ref, w2s):
    @pl.when(pl.program_id(0) == 0)
    def _():
        w2s[...] = sw2_ref[...].astype(jnp.bfloat16)

    y_ref[...] = lax.dot_general(act_ref[...], w2s[...], (((1,), (1,)), ((), ())),
                                 preferred_element_type=jnp.float32)


def _shared(x, sw1, sw3, sw2):
    st = 512
    n = T // st
    act = pl.pallas_call(
        _shared_up_body,
        grid=(n,),
        in_specs=[
            pl.BlockSpec((st, D_MODEL), lambda i: (i, 0)),
            pl.BlockSpec((SHARED_HIDDEN, D_MODEL), lambda i: (0, 0)),
            pl.BlockSpec((SHARED_HIDDEN, D_MODEL), lambda i: (0, 0)),
        ],
        out_specs=pl.BlockSpec((st, SHARED_HIDDEN), lambda i: (i, 0)),
        out_shape=jax.ShapeDtypeStruct((T, SHARED_HIDDEN), jnp.bfloat16),
        scratch_shapes=[pltpu.VMEM((SHARED_HIDDEN, D_MODEL), jnp.bfloat16),
                        pltpu.VMEM((SHARED_HIDDEN, D_MODEL), jnp.bfloat16)],
    )(x, sw1, sw3)
    return pl.pallas_call(
        _shared_down_body,
        grid=(n,),
        in_specs=[
            pl.BlockSpec((st, SHARED_HIDDEN), lambda i: (i, 0)),
            pl.BlockSpec((D_MODEL, SHARED_HIDDEN), lambda i: (0, 0)),
        ],
        out_specs=pl.BlockSpec((st, D_MODEL), lambda i: (i, 0)),
        out_shape=jax.ShapeDtypeStruct((T, D_MODEL), jnp.float32),
        scratch_shapes=[pltpu.VMEM((D_MODEL, SHARED_HIDDEN), jnp.bfloat16)],
    )(act, sw2)


CB_CH = 8
CB_NCH = TOK_PER_W // CB_CH


def _combine_body(ob_hbm, ysh_hbm, w0_hbm, w1_hbm, c0_hbm, c1_hbm, y_hbm,
                  g0a, g0b, g1a, g1b, acca, accb, wv0a, wv0b, wv1a, wv1b,
                  i0all, i1all,
                  sg0a, sg0b, sg1a, sg1b, saa, sab, swba, swbb):
    wid = lax.axis_index("s") * 2 + lax.axis_index("c")
    g0 = [g0a, g0b]
    g1 = [g1a, g1b]
    acc = [acca, accb]
    wv0 = [wv0a, wv0b]
    wv1 = [wv1a, wv1b]
    sg0 = [sg0a, sg0b]
    sg1 = [sg1a, sg1b]
    sa = [saa, sab]
    swb = [swba, swbb]
    pltpu.sync_copy(c0_hbm.at[wid], i0all)
    pltpu.sync_copy(c1_hbm.at[wid], i1all)

    def gth0(cc, b):
        return pltpu.make_async_copy(ob_hbm.at[i0all.at[cc]], g0[b], sg0[b])

    def gth1(cc, b):
        return pltpu.make_async_copy(ob_hbm.at[i1all.at[cc]], g1[b], sg1[b])

    def aux(cc, b):
        base = wid * TOK_PER_W + cc * CB_CH
        return (
            pltpu.make_async_copy(ysh_hbm.at[pl.ds(base, CB_CH)], acc[b], sa[b]),
            pltpu.make_async_copy(w0_hbm.at[wid, cc], wv0[b], sa[b]),
            pltpu.make_async_copy(w1_hbm.at[wid, cc], wv1[b], sa[b]),
        )

    def wb(cc, b):
        base = wid * TOK_PER_W + cc * CB_CH
        return pltpu.make_async_copy(acc[b], y_hbm.at[pl.ds(base, CB_CH)],
                                     swb[b])

    def issue(cc, b):
        gth0(cc, b).start()
        gth1(cc, b).start()
        for d in aux(cc, b):
            d.start()

    issue(0, 0)

    def outer(g, _):
        for b in range(2):
            cc = 2 * g + b
            b1 = 1 - b

            @pl.when(cc + 1 < CB_NCH)
            def _():
                @pl.when(cc >= 1)
                def _():
                    wb(cc - 1, b1).wait()

                issue(cc + 1, b1)

            gth0(cc, b).wait()
            gth1(cc, b).wait()
            for d in aux(cc, b):
                d.wait()
            for r in range(CB_CH):
                s0 = wv0[b][r, :]
                s1 = wv1[b][r, :]

                def col(v, _, r=r, s0=s0, s1=s1, b=b):
                    sl = pl.ds(v * 16, 16)
                    acc[b][r, sl] = (acc[b][r, sl] + s0 * g0[b][r, sl]
                                     + s1 * g1[b][r, sl])
                    return ()

                lax.fori_loop(0, D_MODEL // 16, col, ())
            wb(cc, b).start()
        return ()

    lax.fori_loop(0, CB_NCH // 2, outer, ())
    wb(CB_NCH - 2, 0).wait()
    wb(CB_NCH - 1, 1).wait()


def _combine(ob, ysh, w0, w1, c0, c1):
    mesh = plsc.VectorSubcoreMesh(core_axis_name="c", subcore_axis_name="s")
    f = pl.kernel(
        _combine_body,
        out_type=jax.ShapeDtypeStruct((T, D_MODEL), jnp.float32),
        mesh=mesh,
        scratch_types=[pltpu.VMEM((CB_CH, D_MODEL), jnp.float32)] * 6
        + [pltpu.VMEM((CB_CH, 16), jnp.float32)] * 4
        + [pltpu.VMEM((CB_NCH, CB_CH), jnp.int32)] * 2
        + [pltpu.SemaphoreType.DMA] * 8,
    )
    return f(ob, ysh,
             w0.reshape(NW, CB_NCH, CB_CH, 16),
             w1.reshape(NW, CB_NCH, CB_CH, 16),
             c0.reshape(NW, CB_NCH, CB_CH),
             c1.reshape(NW, CB_NCH, CB_CH))


def kernel(x, gate_w, w1, w3, w2, sw1, sw3, sw2):
    rw0, rw1, c0, c1, d0, d1 = _route(x, gate_w)
    buf = _dispatch(x, d0, d1)
    ob = _experts(buf, w1, w3, w2)
    ysh = _shared(x, sw1, sw3, sw2)
    return _combine(ob, ysh, rw0, rw1, c0, c1)


# speedup vs baseline: 1.0338x; 1.0338x over previous
import functools

import jax
import jax.numpy as jnp
from jax import lax
from jax.experimental import pallas as pl
from jax.experimental.pallas import tpu as pltpu
from jax.experimental.pallas import tpu_sc as plsc

T = 4096
D_MODEL = 2048
HIDDEN = 1024
E = 16
TOPK = 2
SHARED_HIDDEN = 1024
CAPACITY = (T * TOPK // E) * 2
NROWS = E * CAPACITY
DUMP = NROWS
BUF_ROWS = NROWS + 8

TT = 512
BC = 128

NW = 32
TOK_PER_W = T // NW
DISP_CH = 32
COMB_CH = 8


def _route_body(x_ref, gw_ref, w0_ref, w1_ref, c0_ref, c1_ref, d0_ref, d1_ref,
                counts_ref):
    i = pl.program_id(0)

    @pl.when(i == 0)
    def _():
        counts_ref[...] = jnp.zeros_like(counts_ref)

    x = x_ref[...]
    gw = gw_ref[...]
    logits = lax.dot_general(x, gw, (((1,), (1,)), ((), ())),
                             preferred_element_type=jnp.float32,
                             precision=lax.Precision.DEFAULT)
    mx = jnp.max(logits, axis=1, keepdims=True)
    p = jnp.exp(logits - mx)
    s = p / jnp.sum(p, axis=1, keepdims=True)

    iota = lax.broadcasted_iota(jnp.int32, (TT, E), 1)
    m0 = jnp.max(s, axis=1, keepdims=True)
    a0 = jnp.min(jnp.where(s == m0, iota, E), axis=1)
    oh0 = iota == a0[:, None]
    s1 = jnp.where(oh0, -1.0, s)
    m1 = jnp.max(s1, axis=1, keepdims=True)
    a1 = jnp.min(jnp.where(s1 == m1, iota, E), axis=1)
    oh1 = iota == a1[:, None]

    oh0f = oh0.astype(jnp.float32)
    oh1f = oh1.astype(jnp.float32)
    oh = oh0f + oh1f
    r_i = lax.broadcasted_iota(jnp.int32, (TT, TT), 0)
    c_i = lax.broadcasted_iota(jnp.int32, (TT, TT), 1)
    tri = (r_i > c_i).astype(jnp.float32)
    S = lax.dot_general(tri, oh, (((1,), (0,)), ((), ())),
                        preferred_element_type=jnp.float32)
    base = counts_ref[0, :][None, :]
    tot = S + base
    pos0 = jnp.sum(tot * oh0f, axis=1).astype(jnp.int32)
    pos1 = jnp.sum(tot * oh1f, axis=1).astype(jnp.int32)
    counts_ref[0, :] = counts_ref[0, :] + jnp.sum(oh, axis=0)

    slot0 = jnp.minimum(pos0, CAPACITY - 1)
    slot1 = jnp.minimum(pos1, CAPACITY - 1)
    keep0 = pos0 < CAPACITY
    keep1 = pos1 < CAPACITY
    comb0 = a0 * CAPACITY + slot0
    comb1 = a1 * CAPACITY + slot1
    c0_ref[...] = comb0
    c1_ref[...] = comb1
    d0_ref[...] = jnp.where(keep0, comb0, DUMP)
    d1_ref[...] = jnp.where(keep1, comb1, DUMP)
    w0_ref[...] = jnp.broadcast_to(
        jnp.where(keep0, m0[:, 0], 0.0)[:, None], (TT, 16))
    w1_ref[...] = jnp.broadcast_to(
        jnp.where(keep1, m1[:, 0], 0.0)[:, None], (TT, 16))


def _route(x, gate_w):
    n = T // TT
    outs = (
        jax.ShapeDtypeStruct((T, 16), jnp.float32),
        jax.ShapeDtypeStruct((T, 16), jnp.float32),
        jax.ShapeDtypeStruct((T,), jnp.int32),
        jax.ShapeDtypeStruct((T,), jnp.int32),
        jax.ShapeDtypeStruct((T,), jnp.int32),
        jax.ShapeDtypeStruct((T,), jnp.int32),
    )
    vec_spec = pl.BlockSpec((TT,), lambda i: (i,))
    w_spec = pl.BlockSpec((TT, 16), lambda i: (i, 0))
    return pl.pallas_call(
        _route_body,
        grid=(n,),
        in_specs=[
            pl.BlockSpec((TT, D_MODEL), lambda i: (i, 0)),
            pl.BlockSpec((E, D_MODEL), lambda i: (0, 0)),
        ],
        out_specs=(w_spec, w_spec) + (vec_spec,) * 4,
        out_shape=outs,
        scratch_shapes=[pltpu.VMEM((1, E), jnp.float32)],
    )(x, gate_w)


DP_CH = 16
DP_NCH = TOK_PER_W // DP_CH


def _dispatch_body(x_hbm, d0_hbm, d1_hbm, buf_hbm,
                   xva, xvb, d0all, d1all, sxa, sxb, s0a, s0b, s1a, s1b):
    wid = lax.axis_index("s") * 2 + lax.axis_index("c")
    xv = [xva, xvb]
    sx = [sxa, sxb]
    ss0 = [s0a, s0b]
    ss1 = [s1a, s1b]
    pltpu.sync_copy(d0_hbm.at[wid], d0all)
    pltpu.sync_copy(d1_hbm.at[wid], d1all)

    def cp_in(cc, b):
        base = wid * TOK_PER_W + cc * DP_CH
        return pltpu.make_async_copy(x_hbm.at[pl.ds(base, DP_CH)], xv[b], sx[b])

    def sc0(cc, b):
        return pltpu.make_async_copy(xv[b], buf_hbm.at[d0all.at[cc]], ss0[b])

    def sc1(cc, b):
        return pltpu.make_async_copy(xv[b], buf_hbm.at[d1all.at[cc]], ss1[b])

    cp_in(0, 0).start()

    def outer(g, _):
        for b in range(2):
            cc = 2 * g + b
            b1 = 1 - b

            @pl.when(cc + 1 < DP_NCH)
            def _():
                @pl.when(cc >= 1)
                def _():
                    sc0(cc - 1, b1).wait()
                    sc1(cc - 1, b1).wait()

                cp_in(cc + 1, b1).start()

            cp_in(cc, b).wait()
            sc0(cc, b).start()
            sc1(cc, b).start()
        return ()

    lax.fori_loop(0, DP_NCH // 2, outer, ())
    sc0(DP_NCH - 2, 0).wait()
    sc1(DP_NCH - 2, 0).wait()
    sc0(DP_NCH - 1, 1).wait()
    sc1(DP_NCH - 1, 1).wait()


def _dispatch(x, d0, d1):
    mesh = plsc.VectorSubcoreMesh(core_axis_name="c", subcore_axis_name="s")
    f = pl.kernel(
        _dispatch_body,
        out_type=jax.ShapeDtypeStruct((BUF_ROWS, D_MODEL), jnp.float32),
        mesh=mesh,
        scratch_types=[
            pltpu.VMEM((DP_CH, D_MODEL), jnp.float32),
            pltpu.VMEM((DP_CH, D_MODEL), jnp.float32),
            pltpu.VMEM((DP_NCH, DP_CH), jnp.int32),
            pltpu.VMEM((DP_NCH, DP_CH), jnp.int32),
        ] + [pltpu.SemaphoreType.DMA] * 6,
    )
    return f(x, d0.reshape(NW, DP_NCH, DP_CH), d1.reshape(NW, DP_NCH, DP_CH))


def _experts_up_body(buf_ref, w1_ref, w3_ref, act_ref):
    xb = buf_ref[...].astype(jnp.bfloat16)
    w1b = w1_ref[0].astype(jnp.bfloat16)
    w3b = w3_ref[0].astype(jnp.bfloat16)
    h = lax.dot_general(xb, w1b, (((1,), (1,)), ((), ())),
                        preferred_element_type=jnp.float32)
    u = lax.dot_general(xb, w3b, (((1,), (1,)), ((), ())),
                        preferred_element_type=jnp.float32)
    act_ref[...] = (h * jax.nn.sigmoid(h) * u).astype(jnp.bfloat16)


def _experts_down_body(act_ref, w2_ref, ob_ref):
    w2b = w2_ref[0].astype(jnp.bfloat16)
    ob_ref[...] = lax.dot_general(act_ref[...], w2b, (((1,), (1,)), ((), ())),
                                  preferred_element_type=jnp.float32)


BCA = 256
BCB = 512


def _experts(buf, w1, w3, w2):
    nca = CAPACITY // BCA
    act = pl.pallas_call(
        _experts_up_body,
        grid=(E, nca),
        in_specs=[
            pl.BlockSpec((BCA, D_MODEL), lambda e, c: (e * nca + c, 0)),
            pl.BlockSpec((1, HIDDEN, D_MODEL), lambda e, c: (e, 0, 0)),
            pl.BlockSpec((1, HIDDEN, D_MODEL), lambda e, c: (e, 0, 0)),
        ],
        out_specs=pl.BlockSpec((BCA, HIDDEN), lambda e, c: (e * nca + c, 0)),
        out_shape=jax.ShapeDtypeStruct((NROWS, HIDDEN), jnp.bfloat16),
    )(buf, w1, w3)
    ncb = CAPACITY // BCB
    return pl.pallas_call(
        _experts_down_body,
        grid=(E, ncb),
        in_specs=[
            pl.BlockSpec((BCB, HIDDEN), lambda e, c: (e * ncb + c, 0)),
            pl.BlockSpec((1, D_MODEL, HIDDEN), lambda e, c: (e, 0, 0)),
        ],
        out_specs=pl.BlockSpec((BCB, D_MODEL), lambda e, c: (e * ncb + c, 0)),
        out_shape=jax.ShapeDtypeStruct((NROWS, D_MODEL), jnp.float32),
    )(act, w2)


def _shared_body(x_ref, sw1_ref, sw3_ref, sw2_ref, y_ref):
    x = x_ref[...].astype(jnp.bfloat16)
    sw1b = sw1_ref[...].astype(jnp.bfloat16)
    sw3b = sw3_ref[...].astype(jnp.bfloat16)
    sw2b = sw2_ref[...].astype(jnp.bfloat16)
    h = lax.dot_general(x, sw1b, (((1,), (1,)), ((), ())),
                        preferred_element_type=jnp.float32)
    u = lax.dot_general(x, sw3b, (((1,), (1,)), ((), ())),
                        preferred_element_type=jnp.float32)
    act = (h * jax.nn.sigmoid(h) * u).astype(jnp.bfloat16)
    y_ref[...] = lax.dot_general(act, sw2b, (((1,), (1,)), ((), ())),
                                 preferred_element_type=jnp.float32)


def _shared(x, sw1, sw3, sw2):
    st = 256
    n = T // st
    return pl.pallas_call(
        _shared_body,
        grid=(n,),
        in_specs=[
            pl.BlockSpec((st, D_MODEL), lambda i: (i, 0)),
            pl.BlockSpec((SHARED_HIDDEN, D_MODEL), lambda i: (0, 0)),
            pl.BlockSpec((SHARED_HIDDEN, D_MODEL), lambda i: (0, 0)),
            pl.BlockSpec((D_MODEL, SHARED_HIDDEN), lambda i: (0, 0)),
        ],
        out_specs=pl.BlockSpec((st, D_MODEL), lambda i: (i, 0)),
        out_shape=jax.ShapeDtypeStruct((T, D_MODEL), jnp.float32),
    )(x, sw1, sw3, sw2)


CB_CH = 8
CB_NCH = TOK_PER_W // CB_CH


def _combine_body(ob_hbm, ysh_hbm, w0_hbm, w1_hbm, c0_hbm, c1_hbm, y_hbm,
                  g0a, g0b, g1a, g1b, acca, accb, wv0a, wv0b, wv1a, wv1b,
                  i0all, i1all,
                  sg0a, sg0b, sg1a, sg1b, saa, sab, swba, swbb):
    wid = lax.axis_index("s") * 2 + lax.axis_index("c")
    g0 = [g0a, g0b]
    g1 = [g1a, g1b]
    acc = [acca, accb]
    wv0 = [wv0a, wv0b]
    wv1 = [wv1a, wv1b]
    sg0 = [sg0a, sg0b]
    sg1 = [sg1a, sg1b]
    sa = [saa, sab]
    swb = [swba, swbb]
    pltpu.sync_copy(c0_hbm.at[wid], i0all)
    pltpu.sync_copy(c1_hbm.at[wid], i1all)

    def gth0(cc, b):
        return pltpu.make_async_copy(ob_hbm.at[i0all.at[cc]], g0[b], sg0[b])

    def gth1(cc, b):
        return pltpu.make_async_copy(ob_hbm.at[i1all.at[cc]], g1[b], sg1[b])

    def aux(cc, b):
        base = wid * TOK_PER_W + cc * CB_CH
        return (
            pltpu.make_async_copy(ysh_hbm.at[pl.ds(base, CB_CH)], acc[b], sa[b]),
            pltpu.make_async_copy(w0_hbm.at[wid, cc], wv0[b], sa[b]),
            pltpu.make_async_copy(w1_hbm.at[wid, cc], wv1[b], sa[b]),
        )

    def wb(cc, b):
        base = wid * TOK_PER_W + cc * CB_CH
        return pltpu.make_async_copy(acc[b], y_hbm.at[pl.ds(base, CB_CH)],
                                     swb[b])

    def issue(cc, b):
        gth0(cc, b).start()
        gth1(cc, b).start()
        for d in aux(cc, b):
            d.start()

    issue(0, 0)

    def outer(g, _):
        for b in range(2):
            cc = 2 * g + b
            b1 = 1 - b

            @pl.when(cc + 1 < CB_NCH)
            def _():
                @pl.when(cc >= 1)
                def _():
                    wb(cc - 1, b1).wait()

                issue(cc + 1, b1)

            gth0(cc, b).wait()
            gth1(cc, b).wait()
            for d in aux(cc, b):
                d.wait()
            for r in range(CB_CH):
                s0 = wv0[b][r, :]
                s1 = wv1[b][r, :]

                def col(v, _, r=r, s0=s0, s1=s1, b=b):
                    sl = pl.ds(v * 16, 16)
                    acc[b][r, sl] = (acc[b][r, sl] + s0 * g0[b][r, sl]
                                     + s1 * g1[b][r, sl])
                    return ()

                lax.fori_loop(0, D_MODEL // 16, col, ())
            wb(cc, b).start()
        return ()

    lax.fori_loop(0, CB_NCH // 2, outer, ())
    wb(CB_NCH - 2, 0).wait()
    wb(CB_NCH - 1, 1).wait()


def _combine(ob, ysh, w0, w1, c0, c1):
    mesh = plsc.VectorSubcoreMesh(core_axis_name="c", subcore_axis_name="s")
    f = pl.kernel(
        _combine_body,
        out_type=jax.ShapeDtypeStruct((T, D_MODEL), jnp.float32),
        mesh=mesh,
        scratch_types=[pltpu.VMEM((CB_CH, D_MODEL), jnp.float32)] * 6
        + [pltpu.VMEM((CB_CH, 16), jnp.float32)] * 4
        + [pltpu.VMEM((CB_NCH, CB_CH), jnp.int32)] * 2
        + [pltpu.SemaphoreType.DMA] * 8,
    )
    return f(ob, ysh,
             w0.reshape(NW, CB_NCH, CB_CH, 16),
             w1.reshape(NW, CB_NCH, CB_CH, 16),
             c0.reshape(NW, CB_NCH, CB_CH),
             c1.reshape(NW, CB_NCH, CB_CH))


def kernel(x, gate_w, w1, w3, w2, sw1, sw3, sw2):
    rw0, rw1, c0, c1, d0, d1 = _route(x, gate_w)
    buf = _dispatch(x, d0, d1)
    ob = _experts(buf, w1, w3, w2)
    ysh = _shared(x, sw1, sw3, sw2)
    return _combine(ob, ysh, rw0, rw1, c0, c1)
